# baseline (speedup 1.0000x reference)
import functools

import jax
import jax.numpy as jnp
from jax import lax
from jax.experimental import pallas as pl
from jax.experimental.pallas import tpu as pltpu
from jax.experimental.pallas import tpu_sc as plsc

B = 8
N_IN = 196608
N_OUT = 49152
K = 4
C_IN = 16
FILTERS = 32
NK = N_OUT * K
D = B * C_IN

NC = 2
NS = 16
NW = NC * NS
PER_W = NK // NW
IDX_MINOR = 128
IDX_ROWS = PER_W // IDX_MINOR
CHUNK_ROWS = 2
CHUNK = CHUNK_ROWS * IDX_MINOR
NCHUNK = IDX_ROWS // CHUNK_ROWS

_sc_mesh = plsc.VectorSubcoreMesh(core_axis_name="c", subcore_axis_name="s")


@functools.partial(
    pl.kernel,
    out_type=jax.ShapeDtypeStruct((NK, D), jnp.float32),
    mesh=_sc_mesh,
    scratch_types=[
        pltpu.VMEM((IDX_ROWS, IDX_MINOR), jnp.int32),
        pltpu.VMEM((CHUNK, D), jnp.float32),
        pltpu.SemaphoreType.DMA,
    ],
)
def _sc_gather(xt_hbm, idx_hbm, z_hbm, idx_v, rows_v, sem):
    wid = lax.axis_index("s") * NC + lax.axis_index("c")
    pltpu.sync_copy(idx_hbm.at[wid], idx_v)

    def step(ch, carry):
        for j in range(CHUNK_ROWS):
            pltpu.async_copy(
                xt_hbm.at[idx_v.at[ch * CHUNK_ROWS + j]],
                rows_v.at[pl.ds(j * IDX_MINOR, IDX_MINOR)],
                sem,
            )
        pltpu.make_async_copy(z_hbm.at[pl.ds(0, CHUNK)], rows_v, sem).wait()
        pltpu.sync_copy(rows_v, z_hbm.at[pl.ds(wid * PER_W + ch * CHUNK, CHUNK)])
        return carry

    lax.fori_loop(0, NCHUNK, step, 0)


TILE = 2048
KW = K * D
NF = B * FILTERS


def _mm_body(z_ref, w_ref, bias_ref, o_ref):
    y = (
        jnp.dot(z_ref[...], w_ref[...], preferred_element_type=jnp.float32)
        + bias_ref[...]
    )
    for b in range(B):
        o_ref[b] = y[:, b * FILTERS:(b + 1) * FILTERS]


_matmul = pl.pallas_call(
    _mm_body,
    grid=(N_OUT // TILE,),
    in_specs=[
        pl.BlockSpec((TILE, KW), lambda i: (i, 0)),
        pl.BlockSpec((KW, NF), lambda i: (0, 0)),
        pl.BlockSpec((1, NF), lambda i: (0, 0)),
    ],
    out_specs=pl.BlockSpec((B, TILE, FILTERS), lambda i: (0, i, 0)),
    out_shape=jax.ShapeDtypeStruct((B, N_OUT, FILTERS), jnp.float32),
)


def kernel(x, connection_indices, kernel, bias):
    xt = x.transpose(1, 0, 2).reshape(N_IN, D)
    idx = connection_indices.astype(jnp.int32).reshape(NW, IDX_ROWS, IDX_MINOR)
    z = _sc_gather(xt, idx)
    w_big = (
        jnp.eye(B, dtype=jnp.float32)[None, :, None, :, None]
        * kernel[:, None, :, None, :]
    ).reshape(KW, NF)
    bias_t = jnp.tile(bias, B).reshape(1, NF)
    return _matmul(z.reshape(N_OUT, KW), w_big, bias_t)


# speedup vs baseline: 1.3398x; 1.3398x over previous
import functools

import jax
import jax.numpy as jnp
from jax import lax
from jax.experimental import pallas as pl
from jax.experimental.pallas import tpu as pltpu
from jax.experimental.pallas import tpu_sc as plsc

B = 8
N_IN = 196608
N_OUT = 49152
K = 4
C_IN = 16
FILTERS = 32
NK = N_OUT * K
D = B * C_IN

NC = 2
NS = 16
NW = NC * NS
PER_W = NK // NW
IDX_MINOR = 128
IDX_ROWS = PER_W // IDX_MINOR
CHUNK_ROWS = 2
CHUNK = CHUNK_ROWS * IDX_MINOR
NCHUNK = IDX_ROWS // CHUNK_ROWS

_sc_mesh = plsc.VectorSubcoreMesh(core_axis_name="c", subcore_axis_name="s")


NBUF = 3
NROUND = NCHUNK // NBUF


@functools.partial(
    pl.kernel,
    out_type=jax.ShapeDtypeStruct((NK, D), jnp.float32),
    mesh=_sc_mesh,
    scratch_types=[
        pltpu.VMEM((IDX_ROWS, IDX_MINOR), jnp.int32),
        pltpu.VMEM((NBUF, CHUNK, D), jnp.float32),
        pltpu.SemaphoreType.DMA,
        pltpu.SemaphoreType.DMA,
        pltpu.SemaphoreType.DMA,
        pltpu.SemaphoreType.DMA,
        pltpu.SemaphoreType.DMA,
        pltpu.SemaphoreType.DMA,
    ],
)
def _sc_gather(xt_hbm, idx_hbm, z_hbm, idx_v, rows_v, g0, g1, g2, w0, w1, w2):
    gsem = (g0, g1, g2)
    wsem = (w0, w1, w2)
    wid = lax.axis_index("s") * NC + lax.axis_index("c")
    base = wid * PER_W
    pltpu.sync_copy(idx_hbm.at[wid], idx_v)

    def issue_gather(ch, p):
        for j in range(CHUNK_ROWS):
            pltpu.async_copy(
                xt_hbm.at[idx_v.at[ch * CHUNK_ROWS + j]],
                rows_v.at[p].at[pl.ds(j * IDX_MINOR, IDX_MINOR)],
                gsem[p],
            )

    def wait_gather(p):
        pltpu.make_async_copy(z_hbm.at[pl.ds(0, CHUNK)], rows_v.at[p], gsem[p]).wait()

    def issue_wb(ch, p):
        pltpu.async_copy(rows_v.at[p], z_hbm.at[pl.ds(base + ch * CHUNK, CHUNK)], wsem[p])

    def wait_wb(p):
        pltpu.make_async_copy(z_hbm.at[pl.ds(0, CHUNK)], rows_v.at[p], wsem[p]).wait()

    issue_gather(0, 0)
    issue_gather(1, 1)

    def round_(r, carry):
        for p in range(NBUF):
            ch = r * NBUF + p
            if p == 0:
                @pl.when(r >= 1)
                def _():
                    wait_wb(2)
                issue_gather(ch + 2, 2)
            else:
                @pl.when(ch + 2 < NCHUNK)
                def _():
                    wait_wb(p - 1)
                    issue_gather(ch + 2, p - 1)
            wait_gather(p)
            issue_wb(ch, p)
        return carry

    lax.fori_loop(0, NROUND, round_, 0)
    for p in range(NBUF):
        wait_wb(p)


TILE = 2048
KW = K * D
NF = B * FILTERS


def _mm_body(z_ref, w_ref, bias_ref, o_ref):
    o_ref[...] = (
        jnp.dot(z_ref[...], w_ref[...], preferred_element_type=jnp.float32)
        + bias_ref[...]
    )


_matmul = pl.pallas_call(
    _mm_body,
    grid=(N_OUT // TILE,),
    in_specs=[
        pl.BlockSpec((TILE, KW), lambda i: (i, 0)),
        pl.BlockSpec((KW, NF), lambda i: (0, 0)),
        pl.BlockSpec((1, NF), lambda i: (0, 0)),
    ],
    out_specs=pl.BlockSpec((TILE, NF), lambda i: (i, 0)),
    out_shape=jax.ShapeDtypeStruct((N_OUT, NF), jnp.float32),
)


def kernel(x, connection_indices, kernel, bias):
    xt = x.transpose(1, 0, 2).reshape(N_IN, D)
    idx = connection_indices.astype(jnp.int32).reshape(NW, IDX_ROWS, IDX_MINOR)
    z = _sc_gather(xt, idx)
    w_big = (
        jnp.eye(B, dtype=jnp.float32)[None, :, None, :, None]
        * kernel[:, None, :, None, :]
    ).reshape(KW, NF)
    bias_t = jnp.tile(bias, B).reshape(1, NF)
    y = _matmul(z.reshape(N_OUT, KW), w_big, bias_t)
    return y.reshape(N_OUT, B, FILTERS).transpose(1, 0, 2)


# speedup vs baseline: 1.5027x; 1.1215x over previous
import functools

import jax
import jax.numpy as jnp
from jax import lax
from jax.experimental import pallas as pl
from jax.experimental.pallas import tpu as pltpu
from jax.experimental.pallas import tpu_sc as plsc

B = 8
N_IN = 196608
N_OUT = 49152
K = 4
C_IN = 16
FILTERS = 32
NK = N_OUT * K
D = B * C_IN

NC = 2
NS = 16
NW = NC * NS
PER_W = NK // NW
IDX_MINOR = 128
IDX_ROWS = PER_W // IDX_MINOR
CHUNK_ROWS = 2
CHUNK = CHUNK_ROWS * IDX_MINOR
NCHUNK = IDX_ROWS // CHUNK_ROWS

_sc_mesh = plsc.VectorSubcoreMesh(core_axis_name="c", subcore_axis_name="s")


NBUF = 3
NROUND = NCHUNK // NBUF


@functools.partial(
    pl.kernel,
    out_type=jax.ShapeDtypeStruct((NK, D), jnp.float32),
    mesh=_sc_mesh,
    scratch_types=[
        pltpu.VMEM((IDX_ROWS, IDX_MINOR), jnp.int32),
        pltpu.VMEM((NBUF, CHUNK, D), jnp.float32),
        pltpu.SemaphoreType.DMA,
        pltpu.SemaphoreType.DMA,
        pltpu.SemaphoreType.DMA,
        pltpu.SemaphoreType.DMA,
        pltpu.SemaphoreType.DMA,
        pltpu.SemaphoreType.DMA,
    ],
)
def _sc_gather(xt_hbm, idx_hbm, z_hbm, idx_v, rows_v, g0, g1, g2, w0, w1, w2):
    gsem = (g0, g1, g2)
    wsem = (w0, w1, w2)
    wid = lax.axis_index("s") * NC + lax.axis_index("c")
    base = wid * PER_W
    pltpu.sync_copy(idx_hbm.at[wid], idx_v)

    def issue_gather(ch, p):
        for j in range(CHUNK_ROWS):
            pltpu.async_copy(
                xt_hbm.at[idx_v.at[ch * CHUNK_ROWS + j]],
                rows_v.at[p].at[pl.ds(j * IDX_MINOR, IDX_MINOR)],
                gsem[p],
            )

    def wait_gather(p):
        pltpu.make_async_copy(z_hbm.at[pl.ds(0, CHUNK)], rows_v.at[p], gsem[p]).wait()

    def issue_wb(ch, p):
        pltpu.async_copy(rows_v.at[p], z_hbm.at[pl.ds(base + ch * CHUNK, CHUNK)], wsem[p])

    def wait_wb(p):
        pltpu.make_async_copy(z_hbm.at[pl.ds(0, CHUNK)], rows_v.at[p], wsem[p]).wait()

    issue_gather(0, 0)
    issue_gather(1, 1)

    def round_(r, carry):
        for p in range(NBUF):
            ch = r * NBUF + p
            if p == 0:
                @pl.when(r >= 1)
                def _():
                    wait_wb(2)
                issue_gather(ch + 2, 2)
            else:
                @pl.when(ch + 2 < NCHUNK)
                def _():
                    wait_wb(p - 1)
                    issue_gather(ch + 2, p - 1)
            wait_gather(p)
            issue_wb(ch, p)
        return carry

    lax.fori_loop(0, NROUND, round_, 0)
    for p in range(NBUF):
        wait_wb(p)


TILE = 2048
KW = K * D
NF = B * FILTERS


def _mm_body(z_ref, w_ref, bias_ref, o_ref):
    o_ref[...] = (
        jax.lax.dot_general(
            w_ref[...], z_ref[...],
            dimension_numbers=(((1,), (1,)), ((), ())),
            preferred_element_type=jnp.float32,
        )
        + bias_ref[...]
    )


_matmul = pl.pallas_call(
    _mm_body,
    grid=(N_OUT // TILE,),
    in_specs=[
        pl.BlockSpec((TILE, KW), lambda i: (i, 0)),
        pl.BlockSpec((NF, KW), lambda i: (0, 0)),
        pl.BlockSpec((NF, 1), lambda i: (0, 0)),
    ],
    out_specs=pl.BlockSpec((NF, TILE), lambda i: (0, i)),
    out_shape=jax.ShapeDtypeStruct((NF, N_OUT), jnp.float32),
)


def kernel(x, connection_indices, kernel, bias):
    xt = x.transpose(1, 0, 2).reshape(N_IN, D)
    idx = connection_indices.astype(jnp.int32).reshape(NW, IDX_ROWS, IDX_MINOR)
    z = _sc_gather(xt, idx)
    w_big = (
        jnp.eye(B, dtype=jnp.float32)[None, :, None, :, None]
        * kernel[:, None, :, None, :]
    ).reshape(KW, NF)
    bias_t = jnp.tile(bias, B).reshape(NF, 1)
    y = _matmul(z.reshape(N_OUT, KW), w_big.T, bias_t)
    return y.reshape(B, FILTERS, N_OUT).transpose(0, 2, 1)


# speedup vs baseline: 2.3490x; 1.5632x over previous
import functools

import jax
import jax.numpy as jnp
from jax import lax
from jax.experimental import pallas as pl
from jax.experimental.pallas import tpu as pltpu
from jax.experimental.pallas import tpu_sc as plsc

B = 8
N_IN = 196608
N_OUT = 49152
K = 4
C_IN = 16
FILTERS = 32
NK = N_OUT * K
D = B * C_IN

NC = 2
NS = 16
NW = NC * NS
PER_W = NK // NW
IDX_MINOR = 128
IDX_ROWS = PER_W // IDX_MINOR
CHUNK_ROWS = 2
CHUNK = CHUNK_ROWS * IDX_MINOR
NCHUNK = IDX_ROWS // CHUNK_ROWS

_sc_mesh = plsc.VectorSubcoreMesh(core_axis_name="c", subcore_axis_name="s")


NBUF = 3
NROUND = NCHUNK // NBUF


@functools.partial(
    pl.kernel,
    out_type=jax.ShapeDtypeStruct((NK, D), jnp.float32),
    mesh=_sc_mesh,
    scratch_types=[
        pltpu.VMEM((IDX_ROWS, IDX_MINOR), jnp.int32),
        pltpu.VMEM((NBUF, CHUNK, D), jnp.float32),
        pltpu.SemaphoreType.DMA,
        pltpu.SemaphoreType.DMA,
        pltpu.SemaphoreType.DMA,
        pltpu.SemaphoreType.DMA,
        pltpu.SemaphoreType.DMA,
        pltpu.SemaphoreType.DMA,
    ],
)
def _sc_gather(xt_hbm, idx_hbm, z_hbm, idx_v, rows_v, g0, g1, g2, w0, w1, w2):
    gsem = (g0, g1, g2)
    wsem = (w0, w1, w2)
    wid = lax.axis_index("s") * NC + lax.axis_index("c")
    base = wid * PER_W
    pltpu.sync_copy(idx_hbm.at[wid], idx_v)

    def issue_gather(ch, p):
        for j in range(CHUNK_ROWS):
            pltpu.async_copy(
                xt_hbm.at[idx_v.at[ch * CHUNK_ROWS + j]],
                rows_v.at[p].at[pl.ds(j * IDX_MINOR, IDX_MINOR)],
                gsem[p],
            )

    def wait_gather(p):
        pltpu.make_async_copy(z_hbm.at[pl.ds(0, CHUNK)], rows_v.at[p], gsem[p]).wait()

    def issue_wb(ch, p):
        pltpu.async_copy(rows_v.at[p], z_hbm.at[pl.ds(base + ch * CHUNK, CHUNK)], wsem[p])

    def wait_wb(p):
        pltpu.make_async_copy(z_hbm.at[pl.ds(0, CHUNK)], rows_v.at[p], wsem[p]).wait()

    issue_gather(0, 0)
    issue_gather(1, 1)

    def round_(r, carry):
        for p in range(NBUF):
            ch = r * NBUF + p
            if p == 0:
                @pl.when(r >= 1)
                def _():
                    wait_wb(2)
                issue_gather(ch + 2, 2)
            else:
                @pl.when(ch + 2 < NCHUNK)
                def _():
                    wait_wb(p - 1)
                    issue_gather(ch + 2, p - 1)
            wait_gather(p)
            issue_wb(ch, p)
        return carry

    lax.fori_loop(0, NROUND, round_, 0)
    for p in range(NBUF):
        wait_wb(p)


TILE = 2048
KW = K * D
NF = B * FILTERS


NT = N_OUT // TILE


def _mm_body(z0, z1, z2, z3, w_ref, bias_ref, o_ref):
    acc = bias_ref[...]
    for k, zk in enumerate((z0, z1, z2, z3)):
        acc = acc + jax.lax.dot_general(
            w_ref[:, k * D:(k + 1) * D], zk[...],
            dimension_numbers=(((1,), (1,)), ((), ())),
            preferred_element_type=jnp.float32,
        )
    o_ref[...] = acc


def _z_spec(k):
    return pl.BlockSpec((TILE, D), lambda i, k=k: (k * NT + i, 0))


_matmul = pl.pallas_call(
    _mm_body,
    grid=(NT,),
    in_specs=[
        _z_spec(0), _z_spec(1), _z_spec(2), _z_spec(3),
        pl.BlockSpec((NF, KW), lambda i: (0, 0)),
        pl.BlockSpec((NF, 1), lambda i: (0, 0)),
    ],
    out_specs=pl.BlockSpec((NF, TILE), lambda i: (0, i)),
    out_shape=jax.ShapeDtypeStruct((NF, N_OUT), jnp.float32),
)


def kernel(x, connection_indices, kernel, bias):
    xt = x.transpose(1, 0, 2).reshape(N_IN, D)
    idx = connection_indices.astype(jnp.int32).T.reshape(NW, IDX_ROWS, IDX_MINOR)
    z = _sc_gather(xt, idx)
    w_big = (
        jnp.eye(B, dtype=jnp.float32)[None, :, None, :, None]
        * kernel[:, None, :, None, :]
    ).reshape(KW, NF)
    bias_t = jnp.tile(bias, B).reshape(NF, 1)
    y = _matmul(z, z, z, z, w_big.T, bias_t)
    return y.reshape(B, FILTERS, N_OUT).transpose(0, 2, 1)
